# baseline (speedup 1.0000x reference)
import functools

import jax
import jax.numpy as jnp
from jax import lax
from jax.experimental import pallas as pl
from jax.experimental.pallas import tpu as pltpu
from jax.experimental.pallas import tpu_sc as plsc

B = 8
NV = 6890
R = 75
MV = 40
NR = R * MV
NRP = 3008
KF = 16

NW = 32
PER_W = 1536
TOT = NW * PER_W
HALF = TOT // 2
CH = 128


def _sc_gather(table, idx):
    mesh = plsc.VectorSubcoreMesh(core_axis_name="c", subcore_axis_name="s")

    @functools.partial(
        pl.kernel,
        out_type=jax.ShapeDtypeStruct((TOT, KF), jnp.float32),
        mesh=mesh,
        scratch_types=[
            pltpu.VMEM((PER_W,), jnp.int32),
            pltpu.VMEM((PER_W, KF), jnp.float32),
            pltpu.SemaphoreType.DMA,
        ],
        compiler_params=pltpu.CompilerParams(use_tc_tiling_on_sc=False),
    )
    def gather_kernel(table_hbm, idx_hbm, out_hbm, idx_v, rows_v, sem):
        wid = lax.axis_index("s") * 2 + lax.axis_index("c")
        base = wid * PER_W
        pltpu.sync_copy(idx_hbm.at[pl.ds(base, PER_W)], idx_v)
        for j in range(0, PER_W, CH):
            pltpu.async_copy(
                table_hbm.at[idx_v.at[pl.ds(j, CH)]],
                rows_v.at[pl.ds(j, CH)],
                sem,
            )
        for j in range(0, PER_W, CH):
            pltpu.make_async_copy(
                table_hbm.at[idx_v.at[pl.ds(j, CH)]],
                rows_v.at[pl.ds(j, CH)],
                sem,
            ).wait()
        pltpu.sync_copy(rows_v, out_hbm.at[pl.ds(base, PER_W)])

    return gather_kernel(table, idx)


def _dense_body(g1, g2, g1f, g2f, w1, w2, out, b1aug, b2aug):
    h = pl.program_id(1)

    @pl.when(h == 0)
    def _build_baug():
        for src, dst in ((g1f, b1aug), (g2f, b2aug)):
            coords = src[0].T[:3, :]
            yy = jnp.sum(coords * coords, axis=0, keepdims=True)
            ones = jnp.ones_like(yy)
            zeros = jnp.zeros((KF - 5, NRP), jnp.float32)
            dst[...] = jnp.concatenate([coords, ones, yy, zeros], axis=0)
        out[...] = jnp.zeros_like(out)

    def half(a_ref, baug, w):
        a = a_ref[0]
        ac = a[:, :3]
        xx = jnp.sum(ac * ac, axis=1, keepdims=True)
        ones = jnp.ones_like(xx)
        zeros = jnp.zeros((MV, KF - 5), jnp.float32)
        aaug = jnp.concatenate([-2.0 * ac, xx, ones, zeros], axis=1)
        d2 = jnp.dot(aaug, baug[...], preferred_element_type=jnp.float32)
        d2 = jnp.maximum(d2, 1e-12)
        cmin = jnp.min(d2, axis=0, keepdims=True)
        mask = (w[0] != 0.0).astype(jnp.float32)
        return jnp.sum(cmin * mask)

    contrib = half(g1, b2aug, w1) + half(g2, b1aug, w2)
    out[...] += contrib * (1.0 / MV)


def _dense_call(g1r, g2r, g1f, g2f, w1, w2):
    return pl.pallas_call(
        _dense_body,
        grid=(B, R),
        in_specs=[
            pl.BlockSpec((1, MV, KF), lambda b, h: (b, h, 0)),
            pl.BlockSpec((1, MV, KF), lambda b, h: (b, h, 0)),
            pl.BlockSpec((1, NRP, KF), lambda b, h: (b, 0, 0)),
            pl.BlockSpec((1, NRP, KF), lambda b, h: (b, 0, 0)),
            pl.BlockSpec((1, 1, NRP), lambda b, h: (b * R + h, 0, 0)),
            pl.BlockSpec((1, 1, NRP), lambda b, h: (b * R + h, 0, 0)),
        ],
        out_specs=pl.BlockSpec((1, 1, 128), lambda b, h: (b, 0, 0)),
        out_shape=jax.ShapeDtypeStruct((B, 1, 128), jnp.float32),
        scratch_shapes=[
            pltpu.VMEM((KF, NRP), jnp.float32),
            pltpu.VMEM((KF, NRP), jnp.float32),
        ],
        compiler_params=pltpu.CompilerParams(
            dimension_semantics=("arbitrary", "arbitrary")),
    )(g1r, g2r, g1f, g2f, w1, w2)


def kernel(v1, v2, cmap, rid_to_vid_list):
    f32 = jnp.float32
    v1 = v1.astype(f32)
    v2 = v2.astype(f32)

    t = jnp.concatenate([v1.reshape(B * NV, 3), v2.reshape(B * NV, 3)], axis=0)
    t = jnp.pad(t, ((0, 0), (0, KF - 3)))

    rid = rid_to_vid_list.reshape(-1).astype(jnp.int32)
    boff = (jnp.arange(B, dtype=jnp.int32) * NV)[:, None]
    idx1 = (boff + rid[None, :]).reshape(-1)
    idx1 = jnp.pad(idx1, (0, HALF - NR * B))
    idx2 = idx1 + (B * NV)
    idx = jnp.concatenate([idx1, idx2], axis=0)

    g = _sc_gather(t, idx)
    g1 = g[: B * NR].reshape(B, NR, KF)
    g2 = g[HALF : HALF + B * NR].reshape(B, NR, KF)
    g1f = jnp.pad(g1, ((0, 0), (0, NRP - NR), (0, 0)))
    g2f = jnp.pad(g2, ((0, 0), (0, NRP - NR), (0, 0)))

    w1 = jnp.pad(jnp.repeat(cmap, MV, axis=2), ((0, 0), (0, 0), (0, NRP - NR)))
    w2 = jnp.pad(jnp.repeat(cmap.transpose(0, 2, 1), MV, axis=2),
                 ((0, 0), (0, 0), (0, NRP - NR)))
    w1 = w1.reshape(B * R, 1, NRP)
    w2 = w2.reshape(B * R, 1, NRP)

    out = _dense_call(g1, g2, g1f, g2f, w1, w2)
    return out[:, 0, 0]


# speedup vs baseline: 1.8096x; 1.8096x over previous
import functools

import jax
import jax.numpy as jnp
from jax import lax
from jax.experimental import pallas as pl
from jax.experimental.pallas import tpu as pltpu
from jax.experimental.pallas import tpu_sc as plsc

B = 8
NV = 6890
R = 75
MV = 40
NR = R * MV
NRP = 3008
KF = 16

NW = 32
PER_W = 1536
TOT = NW * PER_W
HALF = TOT // 2
CH = 128


def _sc_gather(table, idx):
    mesh = plsc.VectorSubcoreMesh(core_axis_name="c", subcore_axis_name="s")

    @functools.partial(
        pl.kernel,
        out_type=jax.ShapeDtypeStruct((TOT, KF), jnp.float32),
        mesh=mesh,
        scratch_types=[
            pltpu.VMEM((PER_W,), jnp.int32),
            pltpu.VMEM((PER_W, KF), jnp.float32),
            pltpu.SemaphoreType.DMA,
        ],
        compiler_params=pltpu.CompilerParams(use_tc_tiling_on_sc=False),
    )
    def gather_kernel(table_hbm, idx_hbm, out_hbm, idx_v, rows_v, sem):
        wid = lax.axis_index("s") * 2 + lax.axis_index("c")
        base = wid * PER_W
        pltpu.sync_copy(idx_hbm.at[pl.ds(base, PER_W)], idx_v)
        for j in range(0, PER_W, CH):
            pltpu.async_copy(
                table_hbm.at[idx_v.at[pl.ds(j, CH)]],
                rows_v.at[pl.ds(j, CH)],
                sem,
            )
        for j in range(0, PER_W, CH):
            pltpu.make_async_copy(
                table_hbm.at[idx_v.at[pl.ds(j, CH)]],
                rows_v.at[pl.ds(j, CH)],
                sem,
            ).wait()
        pltpu.sync_copy(rows_v, out_hbm.at[pl.ds(base, PER_W)])

    return gather_kernel(table, idx)


NRG = 5
RT = NRG * MV
NT = R // NRG


def _dense_body(g1, g2, g1f, g2f, w1, w2, out, b1aug, b2aug):
    t = pl.program_id(1)

    @pl.when(t == 0)
    def _build_baug():
        for src, dst in ((g1f, b1aug), (g2f, b2aug)):
            coords = src[0].T[:3, :]
            yy = jnp.sum(coords * coords, axis=0, keepdims=True)
            ones = jnp.ones_like(yy)
            zeros = jnp.zeros((KF - 5, NRP), jnp.float32)
            dst[...] = jnp.concatenate([coords, ones, yy, zeros], axis=0)
        out[...] = jnp.zeros_like(out)

    def half(a_ref, baug, w):
        a = a_ref[0]
        ac = a[:, :3]
        xx = jnp.sum(ac * ac, axis=1, keepdims=True)
        ones = jnp.ones_like(xx)
        zeros = jnp.zeros((RT, KF - 5), jnp.float32)
        aaug = jnp.concatenate([-2.0 * ac, xx, ones, zeros], axis=1)
        d2 = jnp.dot(aaug, baug[...], preferred_element_type=jnp.float32)
        d2 = jnp.maximum(d2, 1e-12)
        cmin = jnp.min(d2.reshape(NRG, MV, NRP), axis=1)
        mask = (w[0] != 0.0).astype(jnp.float32)
        return jnp.sum(cmin * mask)

    contrib = half(g1, b2aug, w1) + half(g2, b1aug, w2)
    out[...] += contrib * (1.0 / MV)


def _dense_call(g1r, g2r, g1f, g2f, w1, w2):
    return pl.pallas_call(
        _dense_body,
        grid=(B, NT),
        in_specs=[
            pl.BlockSpec((1, RT, KF), lambda b, t: (b, t, 0)),
            pl.BlockSpec((1, RT, KF), lambda b, t: (b, t, 0)),
            pl.BlockSpec((1, NRP, KF), lambda b, t: (b, 0, 0)),
            pl.BlockSpec((1, NRP, KF), lambda b, t: (b, 0, 0)),
            pl.BlockSpec((1, NRG, NRP), lambda b, t: (b * NT + t, 0, 0)),
            pl.BlockSpec((1, NRG, NRP), lambda b, t: (b * NT + t, 0, 0)),
        ],
        out_specs=pl.BlockSpec((1, 1, 128), lambda b, t: (b, 0, 0)),
        out_shape=jax.ShapeDtypeStruct((B, 1, 128), jnp.float32),
        scratch_shapes=[
            pltpu.VMEM((KF, NRP), jnp.float32),
            pltpu.VMEM((KF, NRP), jnp.float32),
        ],
        compiler_params=pltpu.CompilerParams(
            dimension_semantics=("arbitrary", "arbitrary")),
    )(g1r, g2r, g1f, g2f, w1, w2)


def kernel(v1, v2, cmap, rid_to_vid_list):
    f32 = jnp.float32
    v1 = v1.astype(f32)
    v2 = v2.astype(f32)

    t = jnp.concatenate([v1.reshape(B * NV, 3), v2.reshape(B * NV, 3)], axis=0)
    t = jnp.pad(t, ((0, 0), (0, KF - 3)))

    rid = rid_to_vid_list.reshape(-1).astype(jnp.int32)
    boff = (jnp.arange(B, dtype=jnp.int32) * NV)[:, None]
    idx1 = (boff + rid[None, :]).reshape(-1)
    idx1 = jnp.pad(idx1, (0, HALF - NR * B))
    idx2 = idx1 + (B * NV)
    idx = jnp.concatenate([idx1, idx2], axis=0)

    g = _sc_gather(t, idx)
    g1 = g[: B * NR].reshape(B, NR, KF)
    g2 = g[HALF : HALF + B * NR].reshape(B, NR, KF)
    g1f = jnp.pad(g1, ((0, 0), (0, NRP - NR), (0, 0)))
    g2f = jnp.pad(g2, ((0, 0), (0, NRP - NR), (0, 0)))

    w1 = jnp.pad(jnp.repeat(cmap, MV, axis=2), ((0, 0), (0, 0), (0, NRP - NR)))
    w2 = jnp.pad(jnp.repeat(cmap.transpose(0, 2, 1), MV, axis=2),
                 ((0, 0), (0, 0), (0, NRP - NR)))
    w1 = w1.reshape(B * NT, NRG, NRP)
    w2 = w2.reshape(B * NT, NRG, NRP)

    out = _dense_call(g1, g2, g1f, g2f, w1, w2)
    return out[:, 0, 0]


# speedup vs baseline: 2.0559x; 1.1361x over previous
import functools

import jax
import jax.numpy as jnp
from jax import lax
from jax.experimental import pallas as pl
from jax.experimental.pallas import tpu as pltpu
from jax.experimental.pallas import tpu_sc as plsc

B = 8
NV = 6890
R = 75
MV = 40
NR = R * MV
NRP = 3008
KF = 16

NW = 32
PER_W = 1536
TOT = NW * PER_W
HALF = TOT // 2
CH = 128


def _sc_gather(table, idx):
    mesh = plsc.VectorSubcoreMesh(core_axis_name="c", subcore_axis_name="s")

    @functools.partial(
        pl.kernel,
        out_type=jax.ShapeDtypeStruct((TOT, KF), jnp.float32),
        mesh=mesh,
        scratch_types=[
            pltpu.VMEM((PER_W,), jnp.int32),
            pltpu.VMEM((PER_W, KF), jnp.float32),
            pltpu.SemaphoreType.DMA,
        ],
        compiler_params=pltpu.CompilerParams(use_tc_tiling_on_sc=False),
    )
    def gather_kernel(table_hbm, idx_hbm, out_hbm, idx_v, rows_v, sem):
        wid = lax.axis_index("s") * 2 + lax.axis_index("c")
        base = wid * PER_W
        pltpu.sync_copy(idx_hbm.at[pl.ds(base, PER_W)], idx_v)
        for j in range(0, PER_W, CH):
            pltpu.async_copy(
                table_hbm.at[idx_v.at[pl.ds(j, CH)]],
                rows_v.at[pl.ds(j, CH)],
                sem,
            )
        for j in range(0, PER_W, CH):
            pltpu.make_async_copy(
                table_hbm.at[idx_v.at[pl.ds(j, CH)]],
                rows_v.at[pl.ds(j, CH)],
                sem,
            ).wait()
        pltpu.sync_copy(rows_v, out_hbm.at[pl.ds(base, PER_W)])

    return gather_kernel(table, idx)


NRG = 15
RT = NRG * MV
NT = R // NRG


def _dense_body(g1, g2, g1f, g2f, w1, w2, out, b1aug, b2aug):
    t = pl.program_id(1)

    @pl.when(t == 0)
    def _build_baug():
        for src, dst in ((g1f, b1aug), (g2f, b2aug)):
            coords = src[0].T[:3, :]
            yy = jnp.sum(coords * coords, axis=0, keepdims=True)
            ones = jnp.ones_like(yy)
            zeros = jnp.zeros((KF - 5, NRP), jnp.float32)
            dst[...] = jnp.concatenate([coords, ones, yy, zeros], axis=0)
        out[...] = jnp.zeros_like(out)

    def half(a_ref, baug, w):
        a = a_ref[0]
        ac = a[:, :3]
        xx = jnp.sum(ac * ac, axis=1, keepdims=True)
        ones = jnp.ones_like(xx)
        zeros = jnp.zeros((RT, KF - 5), jnp.float32)
        aaug = jnp.concatenate([-2.0 * ac, xx, ones, zeros], axis=1)
        d2 = jnp.dot(aaug, baug[...], preferred_element_type=jnp.float32)
        d2 = jnp.maximum(d2, 1e-12)
        cmin = jnp.min(d2.reshape(NRG, MV, NRP), axis=1)
        mask = (w[0] != 0.0).astype(jnp.float32)
        return jnp.sum(cmin * mask)

    contrib = half(g1, b2aug, w1) + half(g2, b1aug, w2)
    out[...] += contrib * (1.0 / MV)


def _dense_call(g1r, g2r, g1f, g2f, w1, w2):
    return pl.pallas_call(
        _dense_body,
        grid=(B, NT),
        in_specs=[
            pl.BlockSpec((1, RT, KF), lambda b, t: (b, t, 0)),
            pl.BlockSpec((1, RT, KF), lambda b, t: (b, t, 0)),
            pl.BlockSpec((1, NRP, KF), lambda b, t: (b, 0, 0)),
            pl.BlockSpec((1, NRP, KF), lambda b, t: (b, 0, 0)),
            pl.BlockSpec((1, NRG, NRP), lambda b, t: (b * NT + t, 0, 0)),
            pl.BlockSpec((1, NRG, NRP), lambda b, t: (b * NT + t, 0, 0)),
        ],
        out_specs=pl.BlockSpec((1, 1, 128), lambda b, t: (b, 0, 0)),
        out_shape=jax.ShapeDtypeStruct((B, 1, 128), jnp.float32),
        scratch_shapes=[
            pltpu.VMEM((KF, NRP), jnp.float32),
            pltpu.VMEM((KF, NRP), jnp.float32),
        ],
        compiler_params=pltpu.CompilerParams(
            dimension_semantics=("arbitrary", "arbitrary")),
    )(g1r, g2r, g1f, g2f, w1, w2)


def kernel(v1, v2, cmap, rid_to_vid_list):
    f32 = jnp.float32
    v1 = v1.astype(f32)
    v2 = v2.astype(f32)

    t = jnp.concatenate([v1.reshape(B * NV, 3), v2.reshape(B * NV, 3)], axis=0)
    t = jnp.pad(t, ((0, 0), (0, KF - 3)))

    rid = rid_to_vid_list.reshape(-1).astype(jnp.int32)
    boff = (jnp.arange(B, dtype=jnp.int32) * NV)[:, None]
    idx1 = (boff + rid[None, :]).reshape(-1)
    idx1 = jnp.pad(idx1, (0, HALF - NR * B))
    idx2 = idx1 + (B * NV)
    idx = jnp.concatenate([idx1, idx2], axis=0)

    g = _sc_gather(t, idx)
    g1 = g[: B * NR].reshape(B, NR, KF)
    g2 = g[HALF : HALF + B * NR].reshape(B, NR, KF)
    g1f = jnp.pad(g1, ((0, 0), (0, NRP - NR), (0, 0)))
    g2f = jnp.pad(g2, ((0, 0), (0, NRP - NR), (0, 0)))

    w1 = jnp.pad(jnp.repeat(cmap, MV, axis=2), ((0, 0), (0, 0), (0, NRP - NR)))
    w2 = jnp.pad(jnp.repeat(cmap.transpose(0, 2, 1), MV, axis=2),
                 ((0, 0), (0, 0), (0, NRP - NR)))
    w1 = w1.reshape(B * NT, NRG, NRP)
    w2 = w2.reshape(B * NT, NRG, NRP)

    out = _dense_call(g1, g2, g1f, g2f, w1, w2)
    return out[:, 0, 0]


# speedup vs baseline: 2.4040x; 1.1693x over previous
import functools

import jax
import jax.numpy as jnp
from jax import lax
from jax.experimental import pallas as pl
from jax.experimental.pallas import tpu as pltpu
from jax.experimental.pallas import tpu_sc as plsc

B = 8
NV = 6890
R = 75
MV = 40
NR = R * MV
NRP = 3008
KF = 4
KA = 8

NW = 32
PER_W = NRP // 2
TOT = NW * PER_W

NRG = 15
RT = NRG * MV
NT = R // NRG


def _sc_gather(table, idx):
    mesh = plsc.VectorSubcoreMesh(core_axis_name="c", subcore_axis_name="s")

    @functools.partial(
        pl.kernel,
        out_type=jax.ShapeDtypeStruct((TOT, KF), jnp.float32),
        mesh=mesh,
        scratch_types=[
            pltpu.VMEM((PER_W,), jnp.int32),
            pltpu.VMEM((PER_W, KF), jnp.float32),
            pltpu.SemaphoreType.DMA,
        ],
        compiler_params=pltpu.CompilerParams(use_tc_tiling_on_sc=False),
    )
    def gather_kernel(table_hbm, idx_hbm, out_hbm, idx_v, rows_v, sem):
        wid = lax.axis_index("s") * 2 + lax.axis_index("c")
        base = wid * PER_W
        pltpu.sync_copy(idx_hbm.at[pl.ds(base, PER_W)], idx_v)
        chunks = [(j, 128) for j in range(0, PER_W - 96, 128)]
        chunks.append((PER_W - 96, 96))
        for j, c in chunks:
            pltpu.async_copy(
                table_hbm.at[idx_v.at[pl.ds(j, c)]],
                rows_v.at[pl.ds(j, c)],
                sem,
            )
        for j, c in chunks:
            pltpu.make_async_copy(
                table_hbm.at[idx_v.at[pl.ds(j, c)]],
                rows_v.at[pl.ds(j, c)],
                sem,
            ).wait()
        pltpu.sync_copy(rows_v, out_hbm.at[pl.ds(base, PER_W)])

    return gather_kernel(table, idx)


def _dense_body(g1, g2, g1f, g2f, w1, w2, out, b1aug, b2aug):
    t = pl.program_id(1)

    @pl.when(t == 0)
    def _build_baug():
        for src, dst in ((g1f, b1aug), (g2f, b2aug)):
            coords = src[0].T[:3, :]
            yy = jnp.sum(coords * coords, axis=0, keepdims=True)
            ones = jnp.ones_like(yy)
            zeros = jnp.zeros((KA - 5, NRP), jnp.float32)
            dst[...] = jnp.concatenate([coords, ones, yy, zeros], axis=0)
        out[...] = jnp.zeros_like(out)

    def half(a_ref, baug, w):
        a = a_ref[0]
        ac = a[:, :3]
        xx = jnp.sum(ac * ac, axis=1, keepdims=True)
        ones = jnp.ones_like(xx)
        zeros = jnp.zeros((RT, KA - 5), jnp.float32)
        aaug = jnp.concatenate([-2.0 * ac, xx, ones, zeros], axis=1)
        d2 = jnp.dot(aaug, baug[...], preferred_element_type=jnp.float32)
        d2 = jnp.maximum(d2, 1e-12)
        cmin = jnp.min(d2.reshape(NRG, MV, NRP), axis=1)
        mask = w[0].astype(jnp.float32)
        return jnp.sum(cmin * mask)

    contrib = half(g1, b2aug, w1) + half(g2, b1aug, w2)
    out[...] += contrib * (1.0 / MV)


def _dense_call(g1f, g2f, w1, w2):
    return pl.pallas_call(
        _dense_body,
        grid=(B, NT),
        in_specs=[
            pl.BlockSpec((1, RT, KF), lambda b, t: (b, t, 0)),
            pl.BlockSpec((1, RT, KF), lambda b, t: (b, t, 0)),
            pl.BlockSpec((1, NRP, KF), lambda b, t: (b, 0, 0)),
            pl.BlockSpec((1, NRP, KF), lambda b, t: (b, 0, 0)),
            pl.BlockSpec((1, NRG, NRP), lambda b, t: (b * NT + t, 0, 0)),
            pl.BlockSpec((1, NRG, NRP), lambda b, t: (b * NT + t, 0, 0)),
        ],
        out_specs=pl.BlockSpec((1, 1, 128), lambda b, t: (b, 0, 0)),
        out_shape=jax.ShapeDtypeStruct((B, 1, 128), jnp.float32),
        scratch_shapes=[
            pltpu.VMEM((KA, NRP), jnp.float32),
            pltpu.VMEM((KA, NRP), jnp.float32),
        ],
        compiler_params=pltpu.CompilerParams(
            dimension_semantics=("arbitrary", "arbitrary")),
    )(g1f, g2f, g1f, g2f, w1, w2)


def kernel(v1, v2, cmap, rid_to_vid_list):
    f32 = jnp.float32
    v1 = v1.astype(f32)
    v2 = v2.astype(f32)

    t = jnp.concatenate([v1.reshape(B * NV, 3), v2.reshape(B * NV, 3)], axis=0)
    t = jnp.pad(t, ((0, 0), (0, KF - 3)))

    rid = rid_to_vid_list.reshape(-1).astype(jnp.int32)
    rid = jnp.pad(rid, (0, NRP - NR))
    boff = (jnp.arange(B, dtype=jnp.int32) * NV)[None, :, None]
    soff = (jnp.arange(2, dtype=jnp.int32) * (B * NV))[:, None, None]
    idx = (rid[None, None, :] + boff + soff).reshape(-1)

    g = _sc_gather(t, idx)
    g4 = g.reshape(2, B, NRP, KF)

    wb = (cmap != 0).astype(jnp.int8)
    w1 = jnp.pad(jnp.repeat(wb, MV, axis=2), ((0, 0), (0, 0), (0, NRP - NR)))
    w2 = jnp.pad(jnp.repeat(wb.transpose(0, 2, 1), MV, axis=2),
                 ((0, 0), (0, 0), (0, NRP - NR)))
    w1 = w1.reshape(B * NT, NRG, NRP)
    w2 = w2.reshape(B * NT, NRG, NRP)

    out = _dense_call(g4[0], g4[1], w1, w2)
    return out[:, 0, 0]
